# baseline (speedup 1.0000x reference)
import functools

import jax
import jax.numpy as jnp
from jax import lax
from jax.experimental import pallas as pl
from jax.experimental.pallas import tpu as pltpu

_NBINS = 100


def _plane_kernel(refs_ref, x_ref, o_ref, hist_ref):
    i = pl.program_id(0)
    c = lax.rem(i, 3)
    is_l = c == 0
    lo = jnp.where(is_l, 0.0, -128.0)
    hi = jnp.where(is_l, 100.0, 127.0)
    width = hi - lo

    x = x_ref[0]
    t = (x - lo) / width * float(_NBINS)
    idxf = jnp.floor(t)
    idxf = jnp.clip(idxf, 0.0, float(_NBINS - 1))
    valid = (x >= lo) & (x <= hi)
    pix_bin = jnp.where(valid, idxf, 1.0e6)

    def hist_body(b, _):
        cnt = jnp.sum(jnp.where(pix_bin == b.astype(jnp.float32), 1.0, 0.0))
        hist_ref[b] = cnt
        return 0

    lax.fori_loop(0, _NBINS, hist_body, 0)

    def max_body(b, carry):
        m, mi = carry
        v = hist_ref[b]
        take = v > m
        return jnp.where(take, v, m), jnp.where(take, b, mi)

    m1, i1 = lax.fori_loop(0, _NBINS, max_body, (-1.0, 0))

    def max2_body(b, carry):
        m, mi = carry
        v = hist_ref[b]
        take = (v > m) & (b != i1)
        return jnp.where(take, v, m), jnp.where(take, b, mi)

    m2, i2 = lax.fori_loop(0, _NBINS, max2_body, (-1.0, 0))

    peak_bin = jnp.where(is_l, i2, i1).astype(jnp.float32)
    bin_size = width / float(_NBINS)
    p = lo + (peak_bin + 0.5) * bin_size
    refv = refs_ref[c]
    s = refv / p
    offs = jnp.where(is_l, 100.0, 128.0)
    denom = jnp.where(is_l, 200.0, 255.0)
    o_ref[0] = (x * s + offs) / denom


@functools.partial(jax.jit, static_argnames=("interpret",))
def kernel(lab, ref_l, ref_a, ref_b, interpret=False):
    B, C, H, W = lab.shape
    n = B * C
    x = lab.reshape(n, H, W)
    refs = jnp.concatenate([ref_l, ref_a, ref_b]).astype(jnp.float32)

    out = pl.pallas_call(
        _plane_kernel,
        grid=(n,),
        in_specs=[
            pl.BlockSpec(memory_space=pltpu.SMEM),
            pl.BlockSpec((1, H, W), lambda i: (i, 0, 0)),
        ],
        out_specs=pl.BlockSpec((1, H, W), lambda i: (i, 0, 0)),
        out_shape=jax.ShapeDtypeStruct((n, H, W), jnp.float32),
        scratch_shapes=[pltpu.SMEM((_NBINS,), jnp.float32)],
        interpret=interpret,
    )(refs, x)
    return out.reshape(B, C, H, W)


# speedup vs baseline: 6.0162x; 6.0162x over previous
import functools

import jax
import jax.numpy as jnp
from jax import lax
from jax.experimental import pallas as pl
from jax.experimental.pallas import tpu as pltpu
from jax.experimental.pallas import tpu_sc as plsc

_NBINS = 100
_NPAD = 128
_H = 512
_W = 512
_PIX = _H * _W
_CHUNK = 32768
_NCHUNK = _PIX // _CHUNK
_UNROLL = 8

_CH_LO = (0.0, -128.0, -128.0)
_CH_HI = (100.0, 127.0, 127.0)


def _sc_hist_body(x_hbm, hist_hbm, buf0, buf1, hist_v, row_v, sem0, sem1):
    nc = 2
    wid = lax.axis_index("s") * nc + lax.axis_index("c")
    bufs = (buf0, buf1)
    sems = (sem0, sem1)

    lane = lax.iota(jnp.int32, 16)
    ones16 = jnp.full((16,), 1.0, dtype=jnp.float32)
    zeros16 = jnp.zeros((16,), dtype=jnp.float32)

    for j in range(16 * (_NPAD + 1) // 16):
        plsc.store_scatter(hist_v, [lane + j * 16], zeros16)

    first = pltpu.async_copy(x_hbm.at[wid * 3, pl.ds(0, _CHUNK)], buf0, sem0)
    handles = [first]

    for k in range(3):
        row = wid * 3 + k
        lo = _CH_LO[k]
        hi = _CH_HI[k]
        width = hi - lo
        for ci in range(_NCHUNK):
            g = k * _NCHUNK + ci
            if g + 1 < 3 * _NCHUNK:
                nk = (g + 1) // _NCHUNK
                nci = (g + 1) % _NCHUNK
                h = pltpu.async_copy(
                    x_hbm.at[wid * 3 + nk, pl.ds(nci * _CHUNK, _CHUNK)],
                    bufs[(g + 1) % 2],
                    sems[(g + 1) % 2],
                )
                handles.append(h)
            handles[g].wait()
            buf = bufs[g % 2]

            def chunk_body(it, _, buf=buf, lo=lo, hi=hi, width=width):
                base = it * (16 * _UNROLL)
                for u in range(_UNROLL):
                    x = buf[pl.ds(base + u * 16, 16)]
                    t = (x - lo) / width * float(_NBINS)
                    i = t.astype(jnp.int32)
                    i = jnp.minimum(jnp.maximum(i, 0), _NBINS - 1)
                    valid = (x >= lo) & (x <= hi)
                    flat = lane * (_NPAD + 1) + i
                    plsc.addupdate_scatter(hist_v, [flat], ones16, mask=valid)
                return 0

            lax.fori_loop(0, _CHUNK // (16 * _UNROLL), chunk_body, 0)

        for j in range(_NPAD // 16):
            s = zeros16
            for r in range(16):
                idx = lane + (r * (_NPAD + 1) + j * 16)
                s = s + plsc.load_gather(hist_v, [idx])
                plsc.store_scatter(hist_v, [idx], zeros16)
            row_v[pl.ds(j * 16, 16)] = s
        pltpu.sync_copy(row_v, hist_hbm.at[row])


def _sc_hist(x):
    n = x.shape[0]
    mesh = plsc.VectorSubcoreMesh(core_axis_name="c", subcore_axis_name="s")
    return pl.kernel(
        _sc_hist_body,
        out_type=jax.ShapeDtypeStruct((n, _NPAD), jnp.float32),
        mesh=mesh,
        compiler_params=pltpu.CompilerParams(needs_layout_passes=False),
        scratch_types=[
            pltpu.VMEM((_CHUNK,), jnp.float32),
            pltpu.VMEM((_CHUNK,), jnp.float32),
            pltpu.VMEM((16 * (_NPAD + 1),), jnp.float32),
            pltpu.VMEM((_NPAD,), jnp.float32),
            pltpu.SemaphoreType.DMA,
            pltpu.SemaphoreType.DMA,
        ],
    )(x)


def _norm_kernel(refs_ref, x_ref, hist_ref, o_ref):
    i = pl.program_id(0)
    c = lax.rem(i, 3)
    is_l = c == 0
    lo = jnp.where(is_l, 0.0, -128.0)
    hi = jnp.where(is_l, 100.0, 127.0)
    width = hi - lo

    h = hist_ref[pl.ds(i, 1), :]
    cols = lax.broadcasted_iota(jnp.int32, (1, _NPAD), 1)
    h = jnp.where(cols < _NBINS, h, -1.0)
    m1 = jnp.max(h)
    i1 = jnp.min(jnp.where(h == m1, cols, 1000))
    h2 = jnp.where(cols == i1, -1.0, h)
    m2 = jnp.max(h2)
    i2 = jnp.min(jnp.where(h2 == m2, cols, 1000))
    peak_bin = jnp.where(is_l, i2, i1).astype(jnp.float32)

    bin_size = width / float(_NBINS)
    p = lo + (peak_bin + 0.5) * bin_size
    s = refs_ref[c] / p
    offs = jnp.where(is_l, 100.0, 128.0)
    denom = jnp.where(is_l, 200.0, 255.0)
    o_ref[0] = (x_ref[0] * s + offs) / denom


@jax.jit
def kernel(lab, ref_l, ref_a, ref_b):
    B, C, H, W = lab.shape
    n = B * C
    x = lab.reshape(n, H, W)
    refs = jnp.concatenate([ref_l, ref_a, ref_b]).astype(jnp.float32)

    hist = _sc_hist(lab.reshape(n, H * W))

    out = pl.pallas_call(
        _norm_kernel,
        grid=(n,),
        in_specs=[
            pl.BlockSpec(memory_space=pltpu.SMEM),
            pl.BlockSpec((1, H, W), lambda i: (i, 0, 0)),
            pl.BlockSpec((n, _NPAD), lambda i: (0, 0)),
        ],
        out_specs=pl.BlockSpec((1, H, W), lambda i: (i, 0, 0)),
        out_shape=jax.ShapeDtypeStruct((n, H, W), jnp.float32),
    )(refs, x, hist)
    return out.reshape(B, C, H, W)


# speedup vs baseline: 17.6542x; 2.9344x over previous
import functools

import jax
import jax.numpy as jnp
from jax import lax
from jax.experimental import pallas as pl
from jax.experimental.pallas import tpu as pltpu
from jax.experimental.pallas import tpu_sc as plsc

_NBINS = 100
_NPAD = 128
_H = 512
_W = 512
_PIX = _H * _W
_CHUNK = 32768
_NCHUNK = _PIX // _CHUNK
_UNROLL = 8

_CH_LO = (0.0, -128.0, -128.0)
_CH_HI = (100.0, 127.0, 127.0)


def _sc_hist_body(x_hbm, hist_hbm, buf0, buf1, hist_v, row_v, sem0, sem1):
    nc = 2
    wid = lax.axis_index("s") * nc + lax.axis_index("c")
    bufs = (buf0, buf1)
    sems = (sem0, sem1)

    lane = lax.iota(jnp.int32, 16)
    lane_base = lane * (_NPAD + 1)
    ones16 = jnp.full((16,), 1.0, dtype=jnp.float32)
    zeros16 = jnp.zeros((16,), dtype=jnp.float32)

    for j in range(16 * (_NPAD + 1) // 16):
        plsc.store_scatter(hist_v, [lane + j * 16], zeros16)

    first = pltpu.async_copy(x_hbm.at[wid * 3, pl.ds(0, _CHUNK)], buf0, sem0)
    handles = [first]

    for k in range(3):
        row = wid * 3 + k
        lo = _CH_LO[k]
        hi = _CH_HI[k]
        width = hi - lo
        for ci in range(_NCHUNK):
            g = k * _NCHUNK + ci
            if g + 1 < 3 * _NCHUNK:
                nk = (g + 1) // _NCHUNK
                nci = (g + 1) % _NCHUNK
                h = pltpu.async_copy(
                    x_hbm.at[wid * 3 + nk, pl.ds(nci * _CHUNK, _CHUNK)],
                    bufs[(g + 1) % 2],
                    sems[(g + 1) % 2],
                )
                handles.append(h)
            handles[g].wait()
            buf = bufs[g % 2]

            @plsc.parallel_loop(0, _CHUNK, 16, unroll=_UNROLL)
            def chunk_body(off, buf=buf, lo=lo, hi=hi, width=width):
                x = buf[pl.ds(off, 16)]
                t = (x - lo) / width * float(_NBINS)
                i = t.astype(jnp.int32)
                valid = (x >= lo) & (x <= hi)
                flat = lane_base + i
                plsc.addupdate_scatter(hist_v, [flat], ones16, mask=valid)

        for j in range(_NPAD // 16):
            s = zeros16
            for r in range(16):
                idx = lane + (r * (_NPAD + 1) + j * 16)
                s = s + plsc.load_gather(hist_v, [idx])
                plsc.store_scatter(hist_v, [idx], zeros16)
            row_v[pl.ds(j * 16, 16)] = s
        pltpu.sync_copy(row_v, hist_hbm.at[row])


def _sc_hist(x):
    n = x.shape[0]
    mesh = plsc.VectorSubcoreMesh(core_axis_name="c", subcore_axis_name="s")
    return pl.kernel(
        _sc_hist_body,
        out_type=jax.ShapeDtypeStruct((n, _NPAD), jnp.float32),
        mesh=mesh,
        compiler_params=pltpu.CompilerParams(needs_layout_passes=False),
        scratch_types=[
            pltpu.VMEM((_CHUNK,), jnp.float32),
            pltpu.VMEM((_CHUNK,), jnp.float32),
            pltpu.VMEM((16 * (_NPAD + 1),), jnp.float32),
            pltpu.VMEM((_NPAD,), jnp.float32),
            pltpu.SemaphoreType.DMA,
            pltpu.SemaphoreType.DMA,
        ],
    )(x)


def _norm_kernel(refs_ref, x_ref, hist_ref, o_ref):
    i = pl.program_id(0)
    c = lax.rem(i, 3)
    is_l = c == 0
    lo = jnp.where(is_l, 0.0, -128.0)
    hi = jnp.where(is_l, 100.0, 127.0)
    width = hi - lo

    h = hist_ref[pl.ds(i, 1), :]
    cols = lax.broadcasted_iota(jnp.int32, (1, _NPAD), 1)
    h = jnp.where(cols < _NBINS, h, -1.0)
    m1 = jnp.max(h)
    i1 = jnp.min(jnp.where(h == m1, cols, 1000))
    h2 = jnp.where(cols == i1, -1.0, h)
    m2 = jnp.max(h2)
    i2 = jnp.min(jnp.where(h2 == m2, cols, 1000))
    peak_bin = jnp.where(is_l, i2, i1).astype(jnp.float32)

    bin_size = width / float(_NBINS)
    p = lo + (peak_bin + 0.5) * bin_size
    s = refs_ref[c] / p
    offs = jnp.where(is_l, 100.0, 128.0)
    denom = jnp.where(is_l, 200.0, 255.0)
    o_ref[0] = (x_ref[0] * s + offs) / denom


@jax.jit
def kernel(lab, ref_l, ref_a, ref_b):
    B, C, H, W = lab.shape
    n = B * C
    x = lab.reshape(n, H, W)
    refs = jnp.concatenate([ref_l, ref_a, ref_b]).astype(jnp.float32)

    hist = _sc_hist(lab.reshape(n, H * W))

    out = pl.pallas_call(
        _norm_kernel,
        grid=(n,),
        in_specs=[
            pl.BlockSpec(memory_space=pltpu.SMEM),
            pl.BlockSpec((1, H, W), lambda i: (i, 0, 0)),
            pl.BlockSpec((n, _NPAD), lambda i: (0, 0)),
        ],
        out_specs=pl.BlockSpec((1, H, W), lambda i: (i, 0, 0)),
        out_shape=jax.ShapeDtypeStruct((n, H, W), jnp.float32),
    )(refs, x, hist)
    return out.reshape(B, C, H, W)


# speedup vs baseline: 22.7797x; 1.2903x over previous
import functools

import jax
import jax.numpy as jnp
from jax import lax
from jax.experimental import pallas as pl
from jax.experimental.pallas import tpu as pltpu
from jax.experimental.pallas import tpu_sc as plsc

_NBINS = 100
_NPAD = 128
_H = 512
_W = 512
_PIX = _H * _W
_CHUNK = 32768
_NCHUNK = _PIX // _CHUNK
_UNROLL = 8

_CH_LO = (0.0, -128.0, -128.0)
_CH_HI = (100.0, 127.0, 127.0)


def _sc_hist_body(x_hbm, hist_hbm, buf0, buf1, hist_v, row_v, sem0, sem1):
    nc = 2
    wid = lax.axis_index("s") * nc + lax.axis_index("c")
    bufs = (buf0, buf1)
    sems = (sem0, sem1)

    lane = lax.iota(jnp.int32, 16)
    lane_base = lane * (_NPAD + 1)
    ones16 = jnp.full((16,), 1.0, dtype=jnp.float32)
    zeros16 = jnp.zeros((16,), dtype=jnp.float32)

    for j in range(16 * (_NPAD + 1) // 16):
        plsc.store_scatter(hist_v, [lane + j * 16], zeros16)

    rows_per_chunk = _CHUNK // _W

    first = pltpu.async_copy(
        x_hbm.at[wid * 3, pl.ds(0, rows_per_chunk), :], buf0, sem0
    )
    handles = [first]

    for k in range(3):
        row = wid * 3 + k
        lo = _CH_LO[k]
        hi = _CH_HI[k]
        width = hi - lo
        for ci in range(_NCHUNK):
            g = k * _NCHUNK + ci
            if g + 1 < 3 * _NCHUNK:
                nk = (g + 1) // _NCHUNK
                nci = (g + 1) % _NCHUNK
                h = pltpu.async_copy(
                    x_hbm.at[wid * 3 + nk, pl.ds(nci * rows_per_chunk, rows_per_chunk), :],
                    bufs[(g + 1) % 2],
                    sems[(g + 1) % 2],
                )
                handles.append(h)
            handles[g].wait()
            buf = bufs[g % 2]

            @plsc.parallel_loop(0, _CHUNK // 16, 1, unroll=_UNROLL)
            def chunk_body(u, buf=buf, lo=lo, hi=hi, width=width):
                r = lax.shift_right_logical(u, 5)
                c0 = lax.shift_left(jnp.bitwise_and(u, 31), 4)
                x = buf[r, pl.ds(c0, 16)]
                t = (x - lo) / width * float(_NBINS)
                i = t.astype(jnp.int32)
                valid = (x >= lo) & (x <= hi)
                flat = lane_base + i
                plsc.addupdate_scatter(hist_v, [flat], ones16, mask=valid)

        for j in range(_NPAD // 16):
            s = zeros16
            for r in range(16):
                idx = lane + (r * (_NPAD + 1) + j * 16)
                s = s + plsc.load_gather(hist_v, [idx])
                plsc.store_scatter(hist_v, [idx], zeros16)
            row_v[pl.ds(j * 16, 16)] = s
        pltpu.sync_copy(row_v, hist_hbm.at[row])


def _sc_hist(x):
    n = x.shape[0]
    mesh = plsc.VectorSubcoreMesh(core_axis_name="c", subcore_axis_name="s")
    return pl.kernel(
        _sc_hist_body,
        out_type=jax.ShapeDtypeStruct((n, _NPAD), jnp.float32),
        mesh=mesh,
        compiler_params=pltpu.CompilerParams(needs_layout_passes=False),
        scratch_types=[
            pltpu.VMEM((_CHUNK // _W, _W), jnp.float32),
            pltpu.VMEM((_CHUNK // _W, _W), jnp.float32),
            pltpu.VMEM((16 * (_NPAD + 1),), jnp.float32),
            pltpu.VMEM((_NPAD,), jnp.float32),
            pltpu.SemaphoreType.DMA,
            pltpu.SemaphoreType.DMA,
        ],
    )(x)


def _norm_kernel(refs_ref, x_ref, hist_ref, o_ref):
    i = pl.program_id(0)
    c = lax.rem(i, 3)
    is_l = c == 0
    lo = jnp.where(is_l, 0.0, -128.0)
    hi = jnp.where(is_l, 100.0, 127.0)
    width = hi - lo

    h = hist_ref[pl.ds(i, 1), :]
    cols = lax.broadcasted_iota(jnp.int32, (1, _NPAD), 1)
    h = jnp.where(cols < _NBINS, h, -1.0)
    m1 = jnp.max(h)
    i1 = jnp.min(jnp.where(h == m1, cols, 1000))
    h2 = jnp.where(cols == i1, -1.0, h)
    m2 = jnp.max(h2)
    i2 = jnp.min(jnp.where(h2 == m2, cols, 1000))
    peak_bin = jnp.where(is_l, i2, i1).astype(jnp.float32)

    bin_size = width / float(_NBINS)
    p = lo + (peak_bin + 0.5) * bin_size
    s = refs_ref[c] / p
    offs = jnp.where(is_l, 100.0, 128.0)
    denom = jnp.where(is_l, 200.0, 255.0)
    o_ref[0] = (x_ref[0] * s + offs) / denom


@jax.jit
def kernel(lab, ref_l, ref_a, ref_b):
    B, C, H, W = lab.shape
    n = B * C
    x = lab.reshape(n, H, W)
    refs = jnp.concatenate([ref_l, ref_a, ref_b]).astype(jnp.float32)

    hist = _sc_hist(x)

    out = pl.pallas_call(
        _norm_kernel,
        grid=(n,),
        in_specs=[
            pl.BlockSpec(memory_space=pltpu.SMEM),
            pl.BlockSpec((1, H, W), lambda i: (i, 0, 0)),
            pl.BlockSpec((n, _NPAD), lambda i: (0, 0)),
        ],
        out_specs=pl.BlockSpec((1, H, W), lambda i: (i, 0, 0)),
        out_shape=jax.ShapeDtypeStruct((n, H, W), jnp.float32),
    )(refs, x, hist)
    return out.reshape(B, C, H, W)


# speedup vs baseline: 23.0268x; 1.0109x over previous
import functools

import jax
import jax.numpy as jnp
from jax import lax
from jax.experimental import pallas as pl
from jax.experimental.pallas import tpu as pltpu
from jax.experimental.pallas import tpu_sc as plsc

_NBINS = 100
_NPAD = 128
_H = 512
_W = 512
_PIX = _H * _W
_CHUNK = 32768
_NCHUNK = _PIX // _CHUNK
_UNROLL = 8

_CH_LO = (0.0, -128.0, -128.0)
_CH_HI = (100.0, 127.0, 127.0)


def _sc_hist_body(x_hbm, hist_hbm, buf0, buf1, hist_v, row_v, sem0, sem1):
    nc = 2
    wid = lax.axis_index("s") * nc + lax.axis_index("c")
    bufs = (buf0, buf1)
    sems = (sem0, sem1)

    lane = lax.iota(jnp.int32, 16)
    lane_base = lane * (_NPAD + 1)
    ones16 = jnp.full((16,), 1.0, dtype=jnp.float32)
    zeros16 = jnp.zeros((16,), dtype=jnp.float32)

    for j in range(16 * (_NPAD + 1) // 16):
        plsc.store_scatter(hist_v, [lane + j * 16], zeros16)

    rows_per_chunk = _CHUNK // _W

    first = pltpu.async_copy(
        x_hbm.at[wid * 3, pl.ds(0, rows_per_chunk), :], buf0, sem0
    )
    handles = [first]

    for k in range(3):
        row = wid * 3 + k
        lo = _CH_LO[k]
        hi = _CH_HI[k]
        width = hi - lo
        for ci in range(_NCHUNK):
            g = k * _NCHUNK + ci
            if g + 1 < 3 * _NCHUNK:
                nk = (g + 1) // _NCHUNK
                nci = (g + 1) % _NCHUNK
                h = pltpu.async_copy(
                    x_hbm.at[wid * 3 + nk, pl.ds(nci * rows_per_chunk, rows_per_chunk), :],
                    bufs[(g + 1) % 2],
                    sems[(g + 1) % 2],
                )
                handles.append(h)
            handles[g].wait()
            buf = bufs[g % 2]

            @plsc.parallel_loop(0, _CHUNK // 16, 1, unroll=_UNROLL)
            def chunk_body(u, buf=buf, lo=lo, hi=hi, width=width):
                r = lax.shift_right_logical(u, 5)
                c0 = lax.shift_left(jnp.bitwise_and(u, 31), 4)
                x = buf[r, pl.ds(c0, 16)]
                t = (x - lo) / width * float(_NBINS)
                i = t.astype(jnp.int32)
                valid = (x >= lo) & (x <= hi)
                flat = lane_base + i
                plsc.addupdate_scatter(hist_v, [flat], ones16, mask=valid)

        for j in range(_NPAD // 16):
            s = zeros16
            for r in range(16):
                idx = lane + (r * (_NPAD + 1) + j * 16)
                s = s + plsc.load_gather(hist_v, [idx])
                plsc.store_scatter(hist_v, [idx], zeros16)
            row_v[pl.ds(j * 16, 16)] = s
        pltpu.sync_copy(row_v, hist_hbm.at[row])


def _sc_hist(x):
    n = x.shape[0]
    mesh = plsc.VectorSubcoreMesh(core_axis_name="c", subcore_axis_name="s")
    return pl.kernel(
        _sc_hist_body,
        out_type=jax.ShapeDtypeStruct((n, _NPAD), jnp.float32),
        mesh=mesh,
        compiler_params=pltpu.CompilerParams(needs_layout_passes=False),
        scratch_types=[
            pltpu.VMEM((_CHUNK // _W, _W), jnp.float32),
            pltpu.VMEM((_CHUNK // _W, _W), jnp.float32),
            pltpu.VMEM((16 * (_NPAD + 1),), jnp.float32),
            pltpu.VMEM((_NPAD,), jnp.float32),
            pltpu.SemaphoreType.DMA,
            pltpu.SemaphoreType.DMA,
        ],
    )(x)


def _norm_kernel(refs_ref, x_ref, hist_ref, o_ref):
    i = pl.program_id(0)
    c = lax.rem(i, 3)
    is_l = c == 0
    lo = jnp.where(is_l, 0.0, -128.0)
    hi = jnp.where(is_l, 100.0, 127.0)
    width = hi - lo

    h = hist_ref[pl.ds(i, 1), :]
    cols = lax.broadcasted_iota(jnp.int32, (1, _NPAD), 1)
    h = jnp.where(cols < _NBINS, h, -1.0)
    m1 = jnp.max(h)
    i1 = jnp.min(jnp.where(h == m1, cols, 1000))
    h2 = jnp.where(cols == i1, -1.0, h)
    m2 = jnp.max(h2)
    i2 = jnp.min(jnp.where(h2 == m2, cols, 1000))
    peak_bin = jnp.where(is_l, i2, i1).astype(jnp.float32)

    bin_size = width / float(_NBINS)
    p = lo + (peak_bin + 0.5) * bin_size
    offs = jnp.where(is_l, 100.0, 128.0)
    denom = jnp.where(is_l, 200.0, 255.0)
    a = refs_ref[c] / (p * denom)
    b = offs / denom
    o_ref[0] = x_ref[0] * a + b


@jax.jit
def kernel(lab, ref_l, ref_a, ref_b):
    B, C, H, W = lab.shape
    n = B * C
    x = lab.reshape(n, H, W)
    refs = jnp.concatenate([ref_l, ref_a, ref_b]).astype(jnp.float32)

    hist = _sc_hist(x)

    out = pl.pallas_call(
        _norm_kernel,
        grid=(n,),
        in_specs=[
            pl.BlockSpec(memory_space=pltpu.SMEM),
            pl.BlockSpec((1, H, W), lambda i: (i, 0, 0)),
            pl.BlockSpec((n, _NPAD), lambda i: (0, 0)),
        ],
        out_specs=pl.BlockSpec((1, H, W), lambda i: (i, 0, 0)),
        out_shape=jax.ShapeDtypeStruct((n, H, W), jnp.float32),
    )(refs, x, hist)
    return out.reshape(B, C, H, W)


# speedup vs baseline: 27.2671x; 1.1841x over previous
import jax
import jax.numpy as jnp
from jax import lax
from jax.experimental import pallas as pl
from jax.experimental.pallas import tpu as pltpu
from jax.experimental.pallas import tpu_sc as plsc

_NBINS = 100
_STRIDE = 129
_H = 512
_W = 512
_PIX = _H * _W
_CHUNK = 32768
_NCHUNK = _PIX // _CHUNK
_UNROLL = 8

_CH_LO = (0.0, -128.0, -128.0)
_CH_HI = (100.0, 127.0, 127.0)


def _sc_hist_body(x_hbm, peaks_hbm, buf0, buf1, hist_v, out_v, sem0, sem1):
    nc = 2
    wid = lax.axis_index("s") * nc + lax.axis_index("c")
    bufs = (buf0, buf1)
    sems = (sem0, sem1)

    lane = lax.iota(jnp.int32, 16)
    lane_base = lane * _STRIDE
    ones16 = jnp.full((16,), 1.0, dtype=jnp.float32)
    zeros16 = jnp.zeros((16,), dtype=jnp.float32)

    for j in range(16 * _STRIDE // 16):
        plsc.store_scatter(hist_v, [lane + j * 16], zeros16)

    rows_per_chunk = _CHUNK // _W

    first = pltpu.async_copy(
        x_hbm.at[wid * 3, pl.ds(0, rows_per_chunk), :], buf0, sem0
    )
    handles = [first]

    for k in range(3):
        row = wid * 3 + k
        lo = _CH_LO[k]
        hi = _CH_HI[k]
        width = hi - lo
        for ci in range(_NCHUNK):
            g = k * _NCHUNK + ci
            if g + 1 < 3 * _NCHUNK:
                nk = (g + 1) // _NCHUNK
                nci = (g + 1) % _NCHUNK
                h = pltpu.async_copy(
                    x_hbm.at[wid * 3 + nk, pl.ds(nci * rows_per_chunk, rows_per_chunk), :],
                    bufs[(g + 1) % 2],
                    sems[(g + 1) % 2],
                )
                handles.append(h)
            handles[g].wait()
            buf = bufs[g % 2]

            @plsc.parallel_loop(0, _CHUNK // 16, 1, unroll=_UNROLL)
            def chunk_body(u, buf=buf, lo=lo, width=width, k=k):
                r = lax.shift_right_logical(u, 5)
                c0 = lax.shift_left(jnp.bitwise_and(u, 31), 4)
                x = buf[r, pl.ds(c0, 16)]
                t = (x - lo) / width * float(_NBINS)
                i = t.astype(jnp.int32)
                inb = plsc.bitcast(t, jnp.uint32) <= jnp.uint32(0x42C80000)
                flat = lane_base + i
                plsc.addupdate_scatter(hist_v, [flat], ones16, mask=inb)

        svecs = []
        for j in range(8):
            s = zeros16
            for r in range(16):
                idx = lane + (r * _STRIDE + j * 16)
                s = s + plsc.load_gather(hist_v, [idx])
                plsc.store_scatter(hist_v, [idx], zeros16)
            svecs.append(s)

        cols = [lane + j * 16 for j in range(8)]
        sm = [
            jnp.where(cols[j] < _NBINS, svecs[j], -1.0) for j in range(8)
        ]
        m1v = sm[0]
        for j in range(1, 8):
            m1v = jnp.maximum(m1v, sm[j])
        m1 = jnp.max(m1v)
        candv = jnp.full((16,), 1000, dtype=jnp.int32)
        for j in range(8):
            candv = jnp.minimum(candv, jnp.where(sm[j] == m1, cols[j], 1000))
        i1 = jnp.min(candv)

        if k == 0:
            sm2 = [jnp.where(cols[j] == i1, -1.0, sm[j]) for j in range(8)]
            m2v = sm2[0]
            for j in range(1, 8):
                m2v = jnp.maximum(m2v, sm2[j])
            m2 = jnp.max(m2v)
            cand2 = jnp.full((16,), 1000, dtype=jnp.int32)
            for j in range(8):
                cand2 = jnp.minimum(
                    cand2, jnp.where(sm2[j] == m2, cols[j], 1000)
                )
            peak_bin = jnp.min(cand2)
        else:
            peak_bin = i1

        bin_size = width / float(_NBINS)
        binf = jnp.full((16,), peak_bin, dtype=jnp.int32).astype(jnp.float32)
        p = lo + (binf + 0.5) * bin_size
        out_v[...] = p
        pltpu.sync_copy(out_v, peaks_hbm.at[row])


def _sc_hist(x):
    n = x.shape[0]
    mesh = plsc.VectorSubcoreMesh(core_axis_name="c", subcore_axis_name="s")
    return pl.kernel(
        _sc_hist_body,
        out_type=jax.ShapeDtypeStruct((n, 16), jnp.float32),
        mesh=mesh,
        compiler_params=pltpu.CompilerParams(needs_layout_passes=False),
        scratch_types=[
            pltpu.VMEM((_CHUNK // _W, _W), jnp.float32),
            pltpu.VMEM((_CHUNK // _W, _W), jnp.float32),
            pltpu.VMEM((16 * _STRIDE,), jnp.float32),
            pltpu.VMEM((16,), jnp.float32),
            pltpu.SemaphoreType.DMA,
            pltpu.SemaphoreType.DMA,
        ],
    )(x)


def _norm_kernel(refs_ref, peaks_ref, x_ref, o_ref):
    i = pl.program_id(0)
    c = lax.rem(i, 3)
    is_l = c == 0
    p = peaks_ref[i, 0]
    offs = jnp.where(is_l, 100.0, 128.0)
    denom = jnp.where(is_l, 200.0, 255.0)
    a = refs_ref[c] / (p * denom)
    b = offs / denom
    o_ref[0] = x_ref[0] * a + b


@jax.jit
def kernel(lab, ref_l, ref_a, ref_b):
    B, C, H, W = lab.shape
    n = B * C
    x = lab.reshape(n, H, W)
    refs = jnp.concatenate([ref_l, ref_a, ref_b]).astype(jnp.float32)

    peaks = _sc_hist(x)

    out = pl.pallas_call(
        _norm_kernel,
        grid=(n,),
        in_specs=[
            pl.BlockSpec(memory_space=pltpu.SMEM),
            pl.BlockSpec(memory_space=pltpu.SMEM),
            pl.BlockSpec((1, H, W), lambda i: (i, 0, 0)),
        ],
        out_specs=pl.BlockSpec((1, H, W), lambda i: (i, 0, 0)),
        out_shape=jax.ShapeDtypeStruct((n, H, W), jnp.float32),
    )(refs, peaks, x)
    return out.reshape(B, C, H, W)


# speedup vs baseline: 31.4480x; 1.1533x over previous
import jax
import jax.numpy as jnp
from jax import lax
from jax.experimental import pallas as pl
from jax.experimental.pallas import tpu as pltpu
from jax.experimental.pallas import tpu_sc as plsc

_NBINS = 100
_STRIDE = 129
_H = 512
_W = 512
_PIX = _H * _W
_CHUNK = 32768
_NCHUNK = _PIX // _CHUNK
_UNROLL = 8

_CH_LO = (0.0, -128.0, -128.0)
_CH_HI = (100.0, 127.0, 127.0)


def _sc_hist_body(x_hbm, peaks_hbm, buf0, buf1, hist_v, out_v, sem0, sem1):
    nc = 2
    wid = lax.axis_index("s") * nc + lax.axis_index("c")
    bufs = (buf0, buf1)
    sems = (sem0, sem1)

    lane = lax.iota(jnp.int32, 16)
    lane_base = lane * _STRIDE
    ones16 = jnp.full((16,), 1.0, dtype=jnp.float32)
    zeros16 = jnp.zeros((16,), dtype=jnp.float32)

    for j in range(16 * _STRIDE // 16):
        plsc.store_scatter(hist_v, [lane + j * 16], zeros16)

    rows_per_chunk = _CHUNK // _W

    first = pltpu.async_copy(
        x_hbm.at[wid * 3, pl.ds(0, rows_per_chunk), :], buf0, sem0
    )
    handles = [first]

    for k in range(3):
        row = wid * 3 + k
        lo = _CH_LO[k]
        hi = _CH_HI[k]
        width = hi - lo
        for ci in range(_NCHUNK):
            g = k * _NCHUNK + ci
            if g + 1 < 3 * _NCHUNK:
                nk = (g + 1) // _NCHUNK
                nci = (g + 1) % _NCHUNK
                h = pltpu.async_copy(
                    x_hbm.at[wid * 3 + nk, pl.ds(nci * rows_per_chunk, rows_per_chunk), :],
                    bufs[(g + 1) % 2],
                    sems[(g + 1) % 2],
                )
                handles.append(h)
            handles[g].wait()
            buf = bufs[g % 2]

            @plsc.parallel_loop(0, _CHUNK // 16, 1, unroll=_UNROLL)
            def chunk_body(u, buf=buf, lo=lo, width=width, k=k):
                r = lax.shift_right_logical(u, 5)
                c0 = lax.shift_left(jnp.bitwise_and(u, 31), 4)
                x = buf[r, pl.ds(c0, 16)]
                t = (x - lo) / width * float(_NBINS)
                i = t.astype(jnp.int32)
                inb = plsc.bitcast(t, jnp.uint32) <= jnp.uint32(0x42C80000)
                flat = lane_base + i
                plsc.addupdate_scatter(hist_v, [flat], ones16, mask=inb)

        svecs = []
        for j in range(8):
            s = zeros16
            for r in range(16):
                idx = lane + (r * _STRIDE + j * 16)
                s = s + plsc.load_gather(hist_v, [idx])
                plsc.store_scatter(hist_v, [idx], zeros16)
            svecs.append(s)

        cols = [lane + j * 16 for j in range(8)]
        sm = [
            jnp.where(cols[j] < _NBINS, svecs[j], -1.0) for j in range(8)
        ]
        m1v = sm[0]
        for j in range(1, 8):
            m1v = jnp.maximum(m1v, sm[j])
        m1 = jnp.max(m1v)
        candv = jnp.full((16,), 1000, dtype=jnp.int32)
        for j in range(8):
            candv = jnp.minimum(candv, jnp.where(sm[j] == m1, cols[j], 1000))
        i1 = jnp.min(candv)

        if k == 0:
            sm2 = [jnp.where(cols[j] == i1, -1.0, sm[j]) for j in range(8)]
            m2v = sm2[0]
            for j in range(1, 8):
                m2v = jnp.maximum(m2v, sm2[j])
            m2 = jnp.max(m2v)
            cand2 = jnp.full((16,), 1000, dtype=jnp.int32)
            for j in range(8):
                cand2 = jnp.minimum(
                    cand2, jnp.where(sm2[j] == m2, cols[j], 1000)
                )
            peak_bin = jnp.min(cand2)
        else:
            peak_bin = i1

        bin_size = width / float(_NBINS)
        binf = jnp.full((16,), peak_bin, dtype=jnp.int32).astype(jnp.float32)
        p = lo + (binf + 0.5) * bin_size
        out_v[...] = p
        pltpu.sync_copy(out_v, peaks_hbm.at[row])


def _sc_hist(x):
    n = x.shape[0]
    mesh = plsc.VectorSubcoreMesh(core_axis_name="c", subcore_axis_name="s")
    return pl.kernel(
        _sc_hist_body,
        out_type=jax.ShapeDtypeStruct((n, 16), jnp.float32),
        mesh=mesh,
        compiler_params=pltpu.CompilerParams(needs_layout_passes=False),
        scratch_types=[
            pltpu.VMEM((_CHUNK // _W, _W), jnp.float32),
            pltpu.VMEM((_CHUNK // _W, _W), jnp.float32),
            pltpu.VMEM((16 * _STRIDE,), jnp.float32),
            pltpu.VMEM((16,), jnp.float32),
            pltpu.SemaphoreType.DMA,
            pltpu.SemaphoreType.DMA,
        ],
    )(x)


_TCB = 2


def _norm_kernel(refs_ref, peaks_ref, x_ref, o_ref):
    for t in range(_TCB):
        i = pl.program_id(0) * _TCB + t
        c = lax.rem(i, 3)
        is_l = c == 0
        p = peaks_ref[i, 0]
        offs = jnp.where(is_l, 100.0, 128.0)
        denom = jnp.where(is_l, 200.0, 255.0)
        a = refs_ref[c] / (p * denom)
        b = offs / denom
        o_ref[t] = x_ref[t] * a + b


@jax.jit
def kernel(lab, ref_l, ref_a, ref_b):
    B, C, H, W = lab.shape
    n = B * C
    x = lab.reshape(n, H, W)
    refs = jnp.concatenate([ref_l, ref_a, ref_b]).astype(jnp.float32)

    peaks = _sc_hist(x)

    out = pl.pallas_call(
        _norm_kernel,
        grid=(n // _TCB,),
        in_specs=[
            pl.BlockSpec(memory_space=pltpu.SMEM),
            pl.BlockSpec(memory_space=pltpu.SMEM),
            pl.BlockSpec((_TCB, H, W), lambda i: (i, 0, 0)),
        ],
        out_specs=pl.BlockSpec((_TCB, H, W), lambda i: (i, 0, 0)),
        out_shape=jax.ShapeDtypeStruct((n, H, W), jnp.float32),
    )(refs, peaks, x)
    return out.reshape(B, C, H, W)


# speedup vs baseline: 32.9214x; 1.0469x over previous
import jax
import jax.numpy as jnp
from jax import lax
from jax.experimental import pallas as pl
from jax.experimental.pallas import tpu as pltpu
from jax.experimental.pallas import tpu_sc as plsc

_NBINS = 100
_STRIDE = 129
_H = 512
_W = 512
_PIX = _H * _W
_CHUNK = 32768
_NCHUNK = _PIX // _CHUNK
_UNROLL = 8

_CH_LO = (0.0, -128.0, -128.0)
_CH_HI = (100.0, 127.0, 127.0)


def _sc_hist_body(x_hbm, peaks_hbm, buf0, buf1, hist_v, out_v, sem0, sem1):
    nc = 2
    wid = lax.axis_index("s") * nc + lax.axis_index("c")
    bufs = (buf0, buf1)
    sems = (sem0, sem1)

    lane = lax.iota(jnp.int32, 16)
    lane_base = lane * _STRIDE
    ones16 = jnp.full((16,), 1.0, dtype=jnp.float32)
    zeros16 = jnp.zeros((16,), dtype=jnp.float32)

    for j in range(16 * _STRIDE // 16):
        plsc.store_scatter(hist_v, [lane + j * 16], zeros16)

    rows_per_chunk = _CHUNK // _W

    first = pltpu.async_copy(
        x_hbm.at[wid * 3, pl.ds(0, rows_per_chunk), :], buf0, sem0
    )
    handles = [first]

    for k in range(3):
        row = wid * 3 + k
        lo = _CH_LO[k]
        hi = _CH_HI[k]
        width = hi - lo
        for ci in range(_NCHUNK):
            g = k * _NCHUNK + ci
            if g + 1 < 3 * _NCHUNK:
                nk = (g + 1) // _NCHUNK
                nci = (g + 1) % _NCHUNK
                h = pltpu.async_copy(
                    x_hbm.at[wid * 3 + nk, pl.ds(nci * rows_per_chunk, rows_per_chunk), :],
                    bufs[(g + 1) % 2],
                    sems[(g + 1) % 2],
                )
                handles.append(h)
            handles[g].wait()
            buf = bufs[g % 2]

            @plsc.parallel_loop(0, _CHUNK // 16, 1, unroll=_UNROLL)
            def chunk_body(u, buf=buf, lo=lo, width=width, k=k):
                r = lax.shift_right_logical(u, 5)
                c0 = lax.shift_left(jnp.bitwise_and(u, 31), 4)
                x = buf[r, pl.ds(c0, 16)]
                t = (x - lo) / width * float(_NBINS)
                i = t.astype(jnp.int32)
                inb = plsc.bitcast(t, jnp.uint32) <= jnp.uint32(0x42C80000)
                flat = lane_base + i
                plsc.addupdate_scatter(hist_v, [flat], ones16, mask=inb)

        svecs = []
        for j in range(8):
            s = zeros16
            for r in range(16):
                idx = lane + (r * _STRIDE + j * 16)
                s = s + plsc.load_gather(hist_v, [idx])
                plsc.store_scatter(hist_v, [idx], zeros16)
            svecs.append(s)

        cols = [lane + j * 16 for j in range(8)]
        sm = [
            jnp.where(cols[j] < _NBINS, svecs[j], -1.0) for j in range(8)
        ]
        m1v = sm[0]
        for j in range(1, 8):
            m1v = jnp.maximum(m1v, sm[j])
        m1 = jnp.max(m1v)
        candv = jnp.full((16,), 1000, dtype=jnp.int32)
        for j in range(8):
            candv = jnp.minimum(candv, jnp.where(sm[j] == m1, cols[j], 1000))
        i1 = jnp.min(candv)

        if k == 0:
            sm2 = [jnp.where(cols[j] == i1, -1.0, sm[j]) for j in range(8)]
            m2v = sm2[0]
            for j in range(1, 8):
                m2v = jnp.maximum(m2v, sm2[j])
            m2 = jnp.max(m2v)
            cand2 = jnp.full((16,), 1000, dtype=jnp.int32)
            for j in range(8):
                cand2 = jnp.minimum(
                    cand2, jnp.where(sm2[j] == m2, cols[j], 1000)
                )
            peak_bin = jnp.min(cand2)
        else:
            peak_bin = i1

        bin_size = width / float(_NBINS)
        binf = jnp.full((16,), peak_bin, dtype=jnp.int32).astype(jnp.float32)
        p = lo + (binf + 0.5) * bin_size
        out_v[...] = p
        pltpu.sync_copy(out_v, peaks_hbm.at[row])


def _sc_hist(x):
    n = x.shape[0]
    mesh = plsc.VectorSubcoreMesh(core_axis_name="c", subcore_axis_name="s")
    return pl.kernel(
        _sc_hist_body,
        out_type=jax.ShapeDtypeStruct((n, 16), jnp.float32),
        mesh=mesh,
        compiler_params=pltpu.CompilerParams(needs_layout_passes=False),
        scratch_types=[
            pltpu.VMEM((_CHUNK // _W, _W), jnp.float32),
            pltpu.VMEM((_CHUNK // _W, _W), jnp.float32),
            pltpu.VMEM((16 * _STRIDE,), jnp.float32),
            pltpu.VMEM((16,), jnp.float32),
            pltpu.SemaphoreType.DMA,
            pltpu.SemaphoreType.DMA,
        ],
    )(x)


_TCB = 4


def _norm_kernel(refs_ref, peaks_ref, x_ref, o_ref):
    for t in range(_TCB):
        i = pl.program_id(0) * _TCB + t
        c = lax.rem(i, 3)
        is_l = c == 0
        p = peaks_ref[i, 0]
        offs = jnp.where(is_l, 100.0, 128.0)
        denom = jnp.where(is_l, 200.0, 255.0)
        a = refs_ref[c] / (p * denom)
        b = offs / denom
        o_ref[t] = x_ref[t] * a + b


@jax.jit
def kernel(lab, ref_l, ref_a, ref_b):
    B, C, H, W = lab.shape
    n = B * C
    x = lab.reshape(n, H, W)
    refs = jnp.concatenate([ref_l, ref_a, ref_b]).astype(jnp.float32)

    peaks = _sc_hist(x)

    out = pl.pallas_call(
        _norm_kernel,
        grid=(n // _TCB,),
        in_specs=[
            pl.BlockSpec(memory_space=pltpu.SMEM),
            pl.BlockSpec(memory_space=pltpu.SMEM),
            pl.BlockSpec((_TCB, H, W), lambda i: (i, 0, 0)),
        ],
        out_specs=pl.BlockSpec((_TCB, H, W), lambda i: (i, 0, 0)),
        out_shape=jax.ShapeDtypeStruct((n, H, W), jnp.float32),
    )(refs, peaks, x)
    return out.reshape(B, C, H, W)


# speedup vs baseline: 33.2137x; 1.0089x over previous
import jax
import jax.numpy as jnp
from jax import lax
from jax.experimental import pallas as pl
from jax.experimental.pallas import tpu as pltpu
from jax.experimental.pallas import tpu_sc as plsc

_NBINS = 100
_STRIDE = 129
_H = 512
_W = 512
_PIX = _H * _W
_CHUNK = 32768
_NCHUNK = _PIX // _CHUNK
_UNROLL = 8

_CH_LO = (0.0, -128.0, -128.0)
_CH_HI = (100.0, 127.0, 127.0)


def _sc_hist_body(x_hbm, peaks_hbm, buf0, buf1, hist_v, out_v, sem0, sem1):
    nc = 2
    wid = lax.axis_index("s") * nc + lax.axis_index("c")
    bufs = (buf0, buf1)
    sems = (sem0, sem1)

    lane = lax.iota(jnp.int32, 16)
    lane_base = lane * _STRIDE
    ones16 = jnp.full((16,), 1.0, dtype=jnp.float32)
    zeros16 = jnp.zeros((16,), dtype=jnp.float32)

    for j in range(16 * _STRIDE // 16):
        plsc.store_scatter(hist_v, [lane + j * 16], zeros16)

    rows_per_chunk = _CHUNK // _W

    first = pltpu.async_copy(
        x_hbm.at[wid * 3, pl.ds(0, rows_per_chunk), :], buf0, sem0
    )
    handles = [first]

    for k in range(3):
        row = wid * 3 + k
        lo = _CH_LO[k]
        hi = _CH_HI[k]
        width = hi - lo
        for ci in range(_NCHUNK):
            g = k * _NCHUNK + ci
            if g + 1 < 3 * _NCHUNK:
                nk = (g + 1) // _NCHUNK
                nci = (g + 1) % _NCHUNK
                h = pltpu.async_copy(
                    x_hbm.at[wid * 3 + nk, pl.ds(nci * rows_per_chunk, rows_per_chunk), :],
                    bufs[(g + 1) % 2],
                    sems[(g + 1) % 2],
                )
                handles.append(h)
            handles[g].wait()
            buf = bufs[g % 2]

            @plsc.parallel_loop(0, _CHUNK // 16, 1, unroll=_UNROLL)
            def chunk_body(u, buf=buf, lo=lo, width=width, k=k):
                r = lax.shift_right_logical(u, 5)
                c0 = lax.shift_left(jnp.bitwise_and(u, 31), 4)
                x = buf[r, pl.ds(c0, 16)]
                t = (x - lo) / width * float(_NBINS)
                i = t.astype(jnp.int32)
                inb = plsc.bitcast(t, jnp.uint32) <= jnp.uint32(0x42C80000)
                flat = lane_base + i
                plsc.addupdate_scatter(hist_v, [flat], ones16, mask=inb)

        svecs = []
        for j in range(8):
            s = zeros16
            for r in range(16):
                idx = lane + (r * _STRIDE + j * 16)
                s = s + plsc.load_gather(hist_v, [idx])
                plsc.store_scatter(hist_v, [idx], zeros16)
            svecs.append(s)

        cols = [lane + j * 16 for j in range(8)]
        sm = [
            jnp.where(cols[j] < _NBINS, svecs[j], -1.0) for j in range(8)
        ]
        m1v = sm[0]
        for j in range(1, 8):
            m1v = jnp.maximum(m1v, sm[j])
        m1 = jnp.max(m1v)
        candv = jnp.full((16,), 1000, dtype=jnp.int32)
        for j in range(8):
            candv = jnp.minimum(candv, jnp.where(sm[j] == m1, cols[j], 1000))
        i1 = jnp.min(candv)

        if k == 0:
            sm2 = [jnp.where(cols[j] == i1, -1.0, sm[j]) for j in range(8)]
            m2v = sm2[0]
            for j in range(1, 8):
                m2v = jnp.maximum(m2v, sm2[j])
            m2 = jnp.max(m2v)
            cand2 = jnp.full((16,), 1000, dtype=jnp.int32)
            for j in range(8):
                cand2 = jnp.minimum(
                    cand2, jnp.where(sm2[j] == m2, cols[j], 1000)
                )
            peak_bin = jnp.min(cand2)
        else:
            peak_bin = i1

        bin_size = width / float(_NBINS)
        binf = jnp.full((16,), peak_bin, dtype=jnp.int32).astype(jnp.float32)
        p = lo + (binf + 0.5) * bin_size
        out_v[...] = p
        pltpu.sync_copy(out_v, peaks_hbm.at[row])


def _sc_hist(x):
    n = x.shape[0]
    mesh = plsc.VectorSubcoreMesh(core_axis_name="c", subcore_axis_name="s")
    return pl.kernel(
        _sc_hist_body,
        out_type=jax.ShapeDtypeStruct((n, 16), jnp.float32),
        mesh=mesh,
        compiler_params=pltpu.CompilerParams(needs_layout_passes=False),
        scratch_types=[
            pltpu.VMEM((_CHUNK // _W, _W), jnp.float32),
            pltpu.VMEM((_CHUNK // _W, _W), jnp.float32),
            pltpu.VMEM((16 * _STRIDE,), jnp.float32),
            pltpu.VMEM((16,), jnp.float32),
            pltpu.SemaphoreType.DMA,
            pltpu.SemaphoreType.DMA,
        ],
    )(x)


_TCB = 8


def _norm_kernel(refs_ref, peaks_ref, x_ref, o_ref):
    for t in range(_TCB):
        i = pl.program_id(0) * _TCB + t
        c = lax.rem(i, 3)
        is_l = c == 0
        p = peaks_ref[i, 0]
        offs = jnp.where(is_l, 100.0, 128.0)
        denom = jnp.where(is_l, 200.0, 255.0)
        a = refs_ref[c] / (p * denom)
        b = offs / denom
        o_ref[t] = x_ref[t] * a + b


@jax.jit
def kernel(lab, ref_l, ref_a, ref_b):
    B, C, H, W = lab.shape
    n = B * C
    x = lab.reshape(n, H, W)
    refs = jnp.concatenate([ref_l, ref_a, ref_b]).astype(jnp.float32)

    peaks = _sc_hist(x)

    out = pl.pallas_call(
        _norm_kernel,
        grid=(n // _TCB,),
        in_specs=[
            pl.BlockSpec(memory_space=pltpu.SMEM),
            pl.BlockSpec(memory_space=pltpu.SMEM),
            pl.BlockSpec((_TCB, H, W), lambda i: (i, 0, 0)),
        ],
        out_specs=pl.BlockSpec((_TCB, H, W), lambda i: (i, 0, 0)),
        out_shape=jax.ShapeDtypeStruct((n, H, W), jnp.float32),
    )(refs, peaks, x)
    return out.reshape(B, C, H, W)
